# baseline (speedup 1.0000x reference)
import functools

import jax
import jax.numpy as jnp
from jax import lax
from jax.experimental import pallas as pl
from jax.experimental.pallas import tpu as pltpu
from jax.experimental.pallas import tpu_sc as plsc

_CHUNK = 64
_NBUF = 4
_AHEAD = _NBUF - 1
_L = 16


def _prep_body(usage_ref, sum_ref, out_ref):
    K, D = sum_ref.shape
    b16 = (sum_ref[...] / usage_ref[...]).astype(jnp.bfloat16)
    perm = b16.reshape(K, D // 32, 2, 16).swapaxes(2, 3)
    out_ref[...] = perm.reshape(K, D)


def _normalized_table_packed(cluster_usage, embedding_sum):
    K, D = embedding_sum.shape
    BK = 1024
    b16 = pl.pallas_call(
        _prep_body,
        grid=(K // BK,),
        in_specs=[
            pl.BlockSpec((BK, 1), lambda i: (i, 0)),
            pl.BlockSpec((BK, D), lambda i: (i, 0)),
        ],
        out_specs=pl.BlockSpec((BK, D), lambda i: (i, 0)),
        out_shape=jax.ShapeDtypeStruct((K, D), jnp.bfloat16),
    )(cluster_usage[:, None], embedding_sum)
    return jax.lax.bitcast_convert_type(
        b16.reshape(K, D // 2, 2), jnp.int32)


@functools.cache
def _make_lookup(K, D, B, T, NC, NS):
    N = B * T
    NW = NC * NS
    per_w = N // NW
    nch = per_w // _CHUNK
    w_per_row = T // per_w
    mesh = plsc.VectorSubcoreMesh(core_axis_name="c", subcore_axis_name="s")

    @functools.partial(
        pl.kernel,
        mesh=mesh,
        out_type=jax.ShapeDtypeStruct((B, T, D), jnp.int32),
        scratch_types=[
            pltpu.VMEM((per_w,), jnp.int32),
            pltpu.VMEM((_NBUF, _CHUNK, D // 2), jnp.int32),
            pltpu.VMEM((_NBUF, _CHUNK, D), jnp.int32),
        ] + [pltpu.SemaphoreType.DMA] * (2 * _NBUF),
    )
    def lookup(table_hbm, idx_hbm, out_hbm,
               idx_v, rows_pk, rows_w, g0, g1, g2, g3, s0, s1, s2, s3):
        wid = lax.axis_index("s") * NC + lax.axis_index("c")
        brow = wid // w_per_row
        tcol = (wid % w_per_row) * per_w
        gsem = (g0, g1, g2, g3)
        ssem = (s0, s1, s2, s3)

        def gather_copy(c, b):
            return pltpu.make_async_copy(
                table_hbm.at[idx_v.at[pl.ds(c * _CHUNK, _CHUNK)]],
                rows_pk.at[b], gsem[b])

        def scatter_copy(c, b):
            return pltpu.make_async_copy(
                rows_w.at[b],
                out_hbm.at[brow, pl.ds(tcol + c * _CHUNK, _CHUNK)], ssem[b])

        pltpu.sync_copy(idx_hbm.at[brow, pl.ds(tcol, per_w)], idx_v)
        for c in range(_AHEAD):
            gather_copy(c, c).start()

        def ring_body(i, carry):
            for b in range(_NBUF):
                c = i * _NBUF + b
                gather_copy(0, b).wait()

                def widen_body(r, carry2):
                    for g in range(D // (2 * _L)):
                        x = rows_pk[b, r, pl.ds(_L * g, _L)]
                        rows_w[b, r, pl.ds(2 * _L * g, _L)] = x << 16
                        rows_w[b, r, pl.ds(2 * _L * g + _L, _L)] = (
                            x & jnp.int32(-65536))
                    return carry2
                lax.fori_loop(0, _CHUNK, widen_body, None)

                scatter_copy(c, b).start()

                bn = (b + _AHEAD) % _NBUF

                @pl.when(jnp.logical_and(c + _AHEAD < nch, c >= 1))
                def _():
                    scatter_copy(0, bn).wait()
                    gather_copy(c + _AHEAD, bn).start()

                @pl.when(jnp.logical_and(c + _AHEAD < nch, c < 1))
                def _():
                    gather_copy(c + _AHEAD, bn).start()
            return carry
        lax.fori_loop(0, nch // _NBUF, ring_body, None)

        for b in range(_NBUF):
            scatter_copy(0, b).wait()

    return lookup


def kernel(indices, cluster_usage, embedding_sum):
    K, D = embedding_sum.shape
    B, T = indices.shape
    N = B * T

    info = plsc.get_sparse_core_info()
    NC, NS = info.num_cores, info.num_subcores
    NW = NC * NS
    per_w = N // NW
    assert N % (NW * _CHUNK) == 0 and D % (2 * _L) == 0
    assert (per_w // _CHUNK) % _NBUF == 0 and T % per_w == 0

    table = _normalized_table_packed(cluster_usage, embedding_sum)
    out = _make_lookup(K, D, B, T, NC, NS)(table, indices.astype(jnp.int32))
    return jax.lax.bitcast_convert_type(out, jnp.float32)


# speedup vs baseline: 2.0822x; 2.0822x over previous
import functools

import jax
import jax.numpy as jnp
from jax import lax
from jax.experimental import pallas as pl
from jax.experimental.pallas import tpu as pltpu
from jax.experimental.pallas import tpu_sc as plsc

_CHUNK = 64
_NBUF = 4
_AHEAD = _NBUF - 1
_L = 16


def _prep_body(usage_ref, sum_ref, out_ref):
    out_ref[...] = (sum_ref[...] / usage_ref[...]).astype(jnp.bfloat16)


def _normalized_table_packed(cluster_usage, embedding_sum):
    K, D = embedding_sum.shape
    BK = 1024
    b16 = pl.pallas_call(
        _prep_body,
        grid=(K // BK,),
        in_specs=[
            pl.BlockSpec((BK, 1), lambda i: (i, 0)),
            pl.BlockSpec((BK, D), lambda i: (i, 0)),
        ],
        out_specs=pl.BlockSpec((BK, D), lambda i: (i, 0)),
        out_shape=jax.ShapeDtypeStruct((K, D), jnp.bfloat16),
    )(cluster_usage[:, None], embedding_sum)
    perm = b16.reshape(K, D // 32, 2, 16).swapaxes(2, 3)
    return jax.lax.bitcast_convert_type(
        perm.reshape(K, D // 2, 2), jnp.int32)


@functools.cache
def _make_lookup(K, D, B, T, NC, NS):
    N = B * T
    NW = NC * NS
    per_w = N // NW
    nch = per_w // _CHUNK
    w_per_row = T // per_w
    mesh = plsc.VectorSubcoreMesh(core_axis_name="c", subcore_axis_name="s")

    @functools.partial(
        pl.kernel,
        mesh=mesh,
        out_type=jax.ShapeDtypeStruct((B, T, D), jnp.int32),
        scratch_types=[
            pltpu.VMEM((per_w,), jnp.int32),
            pltpu.VMEM((_NBUF, _CHUNK, D // 2), jnp.int32),
            pltpu.VMEM((_NBUF, _CHUNK, D), jnp.int32),
        ] + [pltpu.SemaphoreType.DMA] * (2 * _NBUF),
    )
    def lookup(table_hbm, idx_hbm, out_hbm,
               idx_v, rows_pk, rows_w, g0, g1, g2, g3, s0, s1, s2, s3):
        wid = lax.axis_index("s") * NC + lax.axis_index("c")
        brow = wid // w_per_row
        tcol = (wid % w_per_row) * per_w
        gsem = (g0, g1, g2, g3)
        ssem = (s0, s1, s2, s3)

        def gather_copy(c, b):
            return pltpu.make_async_copy(
                table_hbm.at[idx_v.at[pl.ds(c * _CHUNK, _CHUNK)]],
                rows_pk.at[b], gsem[b])

        def scatter_copy(c, b):
            return pltpu.make_async_copy(
                rows_w.at[b],
                out_hbm.at[brow, pl.ds(tcol + c * _CHUNK, _CHUNK)], ssem[b])

        pltpu.sync_copy(idx_hbm.at[brow, pl.ds(tcol, per_w)], idx_v)
        for c in range(_AHEAD):
            gather_copy(c, c).start()

        def ring_body(i, carry):
            for b in range(_NBUF):
                c = i * _NBUF + b
                gather_copy(0, b).wait()

                def widen_body(r, carry2):
                    for g in range(D // (2 * _L)):
                        x = rows_pk[b, r, pl.ds(_L * g, _L)]
                        rows_w[b, r, pl.ds(2 * _L * g, _L)] = x << 16
                        rows_w[b, r, pl.ds(2 * _L * g + _L, _L)] = (
                            x & jnp.int32(-65536))
                    return carry2
                lax.fori_loop(0, _CHUNK, widen_body, None)

                scatter_copy(c, b).start()

                bn = (b + _AHEAD) % _NBUF

                @pl.when(jnp.logical_and(c + _AHEAD < nch, c >= 1))
                def _():
                    scatter_copy(0, bn).wait()
                    gather_copy(c + _AHEAD, bn).start()

                @pl.when(jnp.logical_and(c + _AHEAD < nch, c < 1))
                def _():
                    gather_copy(c + _AHEAD, bn).start()
            return carry
        lax.fori_loop(0, nch // _NBUF, ring_body, None)

        for b in range(_NBUF):
            scatter_copy(0, b).wait()

    return lookup


def kernel(indices, cluster_usage, embedding_sum):
    K, D = embedding_sum.shape
    B, T = indices.shape
    N = B * T

    info = plsc.get_sparse_core_info()
    NC, NS = info.num_cores, info.num_subcores
    NW = NC * NS
    per_w = N // NW
    assert N % (NW * _CHUNK) == 0 and D % (2 * _L) == 0
    assert (per_w // _CHUNK) % _NBUF == 0 and T % per_w == 0

    table = _normalized_table_packed(cluster_usage, embedding_sum)
    out = _make_lookup(K, D, B, T, NC, NS)(table, indices.astype(jnp.int32))
    return jax.lax.bitcast_convert_type(out, jnp.float32)


# speedup vs baseline: 2.3985x; 1.1519x over previous
import functools

import jax
import jax.numpy as jnp
from jax import lax
from jax.experimental import pallas as pl
from jax.experimental.pallas import tpu as pltpu
from jax.experimental.pallas import tpu_sc as plsc

_CHUNK = 64
_NBUF = 4
_AHEAD = _NBUF - 1
_L = 16


def _rne_bf16_bits(x):
    u = jax.lax.bitcast_convert_type(x, jnp.int32)
    return ((u + 0x7FFF + ((u >> 16) & 1)) >> 16) & 0xFFFF


def _prep_body(usage_ref, sum_ref, out_ref):
    D = sum_ref.shape[1]
    emb = sum_ref[...] / usage_ref[...]
    p = jax.lax.broadcasted_iota(jnp.int32, (D, D // 2), 1)
    i = jax.lax.broadcasted_iota(jnp.int32, (D, D // 2), 0)
    col_a = 32 * (p // 16) + p % 16
    sel_lo = (i == col_a).astype(jnp.float32)
    sel_hi = (i == col_a + 16).astype(jnp.float32)
    lo = jnp.dot(emb, sel_lo, preferred_element_type=jnp.float32)
    hi = jnp.dot(emb, sel_hi, preferred_element_type=jnp.float32)
    out_ref[...] = _rne_bf16_bits(lo) | (_rne_bf16_bits(hi) << 16)


def _normalized_table_packed(cluster_usage, embedding_sum):
    K, D = embedding_sum.shape
    BK = 1024
    return pl.pallas_call(
        _prep_body,
        grid=(K // BK,),
        in_specs=[
            pl.BlockSpec((BK, 1), lambda i: (i, 0)),
            pl.BlockSpec((BK, D), lambda i: (i, 0)),
        ],
        out_specs=pl.BlockSpec((BK, D // 2), lambda i: (i, 0)),
        out_shape=jax.ShapeDtypeStruct((K, D // 2), jnp.int32),
    )(cluster_usage[:, None], embedding_sum)


@functools.cache
def _make_lookup(K, D, B, T, NC, NS):
    N = B * T
    NW = NC * NS
    per_w = N // NW
    nch = per_w // _CHUNK
    w_per_row = T // per_w
    mesh = plsc.VectorSubcoreMesh(core_axis_name="c", subcore_axis_name="s")

    @functools.partial(
        pl.kernel,
        mesh=mesh,
        out_type=jax.ShapeDtypeStruct((B, T, D), jnp.int32),
        scratch_types=[
            pltpu.VMEM((per_w,), jnp.int32),
            pltpu.VMEM((_NBUF, _CHUNK, D // 2), jnp.int32),
            pltpu.VMEM((_NBUF, _CHUNK, D), jnp.int32),
        ] + [pltpu.SemaphoreType.DMA] * (2 * _NBUF),
    )
    def lookup(table_hbm, idx_hbm, out_hbm,
               idx_v, rows_pk, rows_w, g0, g1, g2, g3, s0, s1, s2, s3):
        wid = lax.axis_index("s") * NC + lax.axis_index("c")
        brow = wid // w_per_row
        tcol = (wid % w_per_row) * per_w
        gsem = (g0, g1, g2, g3)
        ssem = (s0, s1, s2, s3)

        def gather_copy(c, b):
            return pltpu.make_async_copy(
                table_hbm.at[idx_v.at[pl.ds(c * _CHUNK, _CHUNK)]],
                rows_pk.at[b], gsem[b])

        def scatter_copy(c, b):
            return pltpu.make_async_copy(
                rows_w.at[b],
                out_hbm.at[brow, pl.ds(tcol + c * _CHUNK, _CHUNK)], ssem[b])

        pltpu.sync_copy(idx_hbm.at[brow, pl.ds(tcol, per_w)], idx_v)
        for c in range(_AHEAD):
            gather_copy(c, c).start()

        def ring_body(i, carry):
            for b in range(_NBUF):
                c = i * _NBUF + b
                gather_copy(0, b).wait()

                def widen_body(r, carry2):
                    for g in range(D // (2 * _L)):
                        x = rows_pk[b, r, pl.ds(_L * g, _L)]
                        rows_w[b, r, pl.ds(2 * _L * g, _L)] = x << 16
                        rows_w[b, r, pl.ds(2 * _L * g + _L, _L)] = (
                            x & jnp.int32(-65536))
                    return carry2
                lax.fori_loop(0, _CHUNK, widen_body, None)

                scatter_copy(c, b).start()

                bn = (b + _AHEAD) % _NBUF

                @pl.when(jnp.logical_and(c + _AHEAD < nch, c >= 1))
                def _():
                    scatter_copy(0, bn).wait()
                    gather_copy(c + _AHEAD, bn).start()

                @pl.when(jnp.logical_and(c + _AHEAD < nch, c < 1))
                def _():
                    gather_copy(c + _AHEAD, bn).start()
            return carry
        lax.fori_loop(0, nch // _NBUF, ring_body, None)

        for b in range(_NBUF):
            scatter_copy(0, b).wait()

    return lookup


def kernel(indices, cluster_usage, embedding_sum):
    K, D = embedding_sum.shape
    B, T = indices.shape
    N = B * T

    info = plsc.get_sparse_core_info()
    NC, NS = info.num_cores, info.num_subcores
    NW = NC * NS
    per_w = N // NW
    assert N % (NW * _CHUNK) == 0 and D % (2 * _L) == 0
    assert (per_w // _CHUNK) % _NBUF == 0 and T % per_w == 0

    table = _normalized_table_packed(cluster_usage, embedding_sum)
    out = _make_lookup(K, D, B, T, NC, NS)(table, indices.astype(jnp.int32))
    return jax.lax.bitcast_convert_type(out, jnp.float32)


# speedup vs baseline: 4.5181x; 1.8837x over previous
import functools

import jax
import jax.numpy as jnp
from jax import lax
from jax.experimental import pallas as pl
from jax.experimental.pallas import tpu as pltpu
from jax.experimental.pallas import tpu_sc as plsc

_CHUNK = 128
_NBUF = 2
_L = 16


@functools.cache
def _make_lookup(K, D, B, T, NC, NS):
    N = B * T
    NW = NC * NS
    per_w = N // NW
    nch = per_w // _CHUNK
    w_per_row = T // per_w
    mesh = plsc.VectorSubcoreMesh(core_axis_name="c", subcore_axis_name="s")

    @functools.partial(
        pl.kernel,
        mesh=mesh,
        out_type=jax.ShapeDtypeStruct((B, T, D), jnp.float32),
        scratch_types=[
            pltpu.VMEM((per_w,), jnp.int32),
            pltpu.VMEM((_NBUF, _CHUNK), jnp.float32),
            pltpu.VMEM((_NBUF, _CHUNK, D), jnp.float32),
        ] + [pltpu.SemaphoreType.DMA] * (2 * _NBUF),
    )
    def lookup(sum_hbm, usage_hbm, idx_hbm, out_hbm,
               idx_v, usage_v, rows_v, g0, g1, s0, s1):
        wid = lax.axis_index("s") * NC + lax.axis_index("c")
        brow = wid // w_per_row
        tcol = (wid % w_per_row) * per_w
        gsem = (g0, g1)
        ssem = (s0, s1)

        def gather_rows(c, b):
            return pltpu.make_async_copy(
                sum_hbm.at[idx_v.at[pl.ds(c * _CHUNK, _CHUNK)]],
                rows_v.at[b], gsem[b])

        def gather_usage(c, b):
            return pltpu.make_async_copy(
                usage_hbm.at[idx_v.at[pl.ds(c * _CHUNK, _CHUNK)]],
                usage_v.at[b], gsem[b])

        def start_gathers(c, b):
            gather_rows(c, b).start()
            gather_usage(c, b).start()

        def wait_gathers(b):
            gather_rows(0, b).wait()
            gather_usage(0, b).wait()

        def scatter_copy(c, b):
            return pltpu.make_async_copy(
                rows_v.at[b],
                out_hbm.at[brow, pl.ds(tcol + c * _CHUNK, _CHUNK)], ssem[b])

        pltpu.sync_copy(idx_hbm.at[brow, pl.ds(tcol, per_w)], idx_v)
        for b in range(_NBUF):
            start_gathers(b, b)

        def ring_body(i, carry):
            for b in range(_NBUF):
                c = i * _NBUF + b
                wait_gathers(b)

                def scale_body(g, carry2):
                    r0 = g * _L
                    scales = 1.0 / usage_v[b, pl.ds(r0, _L)]
                    for j in range(_L):
                        for k in range(D // _L):
                            sl = pl.ds(k * _L, _L)
                            rows_v[b, r0 + j, sl] = (
                                rows_v[b, r0 + j, sl] * scales[j])
                    return carry2
                lax.fori_loop(0, _CHUNK // _L, scale_body, None)

                scatter_copy(c, b).start()

                @pl.when(c + _NBUF < nch)
                def _():
                    scatter_copy(0, b).wait()
                    start_gathers(c + _NBUF, b)
            return carry
        lax.fori_loop(0, nch // _NBUF, ring_body, None)

        for b in range(_NBUF):
            scatter_copy(0, b).wait()

    return lookup


def kernel(indices, cluster_usage, embedding_sum):
    K, D = embedding_sum.shape
    B, T = indices.shape
    N = B * T

    info = plsc.get_sparse_core_info()
    NC, NS = info.num_cores, info.num_subcores
    NW = NC * NS
    per_w = N // NW
    assert N % (NW * _CHUNK) == 0 and D % _L == 0
    assert (per_w // _CHUNK) % _NBUF == 0 and T % per_w == 0

    return _make_lookup(K, D, B, T, NC, NS)(
        embedding_sum, cluster_usage, indices.astype(jnp.int32))
